# baseline (speedup 1.0000x reference)
import functools

import jax
import jax.numpy as jnp
from jax import lax
from jax.experimental import pallas as pl
from jax.experimental.pallas import tpu as pltpu
from jax.experimental.pallas import tpu_sc as plsc

N = 10000
E = 320000
D = 128
DH = D // 2
LAYERS = 4
EPS = 1e-05
LN_EPS = 1e-05

NC = 2
NS = 16
NW = NC * NS
CHUNK = 128
NCH = 80
EP = NW * NCH * CHUNK
NPAD = 10240
RPT = NPAD // NS

_mesh = plsc.VectorSubcoreMesh(core_axis_name="c", subcore_axis_name="s")


@functools.partial(
    pl.kernel,
    out_type=(
        jax.ShapeDtypeStruct((NPAD,), jnp.float32),
        jax.ShapeDtypeStruct((NPAD,), jnp.float32),
    ),
    mesh=_mesh,
    scratch_types=[
        pltpu.VMEM((NCH, CHUNK), jnp.int32),
        pltpu.VMEM((CHUNK,), jnp.float32),
        pltpu.VMEM_SHARED((NPAD,), jnp.float32),
    ],
)
def _deg_kernel(dst_hbm, zeros1d_hbm, ones_hbm, d0_hbm, d1_hbm,
                idx_v, ones_v, deg_sp):
    cid = lax.axis_index("c")
    sid = lax.axis_index("s")
    w = cid * NS + sid
    pltpu.sync_copy(zeros1d_hbm, deg_sp.at[pl.ds(sid * RPT, RPT)])
    pltpu.sync_copy(ones_hbm, ones_v)
    pltpu.sync_copy(dst_hbm.at[w], idx_v)
    plsc.subcore_barrier()

    @pl.loop(0, NCH)
    def _(j):
        pltpu.sync_copy(ones_v, deg_sp.at[idx_v.at[j]], add=True)

    plsc.subcore_barrier()

    @pl.when(jnp.logical_and(sid == 0, cid == 0))
    def _():
        pltpu.sync_copy(deg_sp, d0_hbm)

    @pl.when(jnp.logical_and(sid == 0, cid == 1))
    def _():
        pltpu.sync_copy(deg_sp, d1_hbm)


@functools.partial(
    pl.kernel,
    out_type=(
        jax.ShapeDtypeStruct((NPAD, DH), jnp.float32),
        jax.ShapeDtypeStruct((NPAD, DH), jnp.float32),
        jax.ShapeDtypeStruct((NPAD, DH), jnp.float32),
        jax.ShapeDtypeStruct((NPAD, DH), jnp.float32),
    ),
    mesh=_mesh,
    scratch_types=[
        pltpu.VMEM((NCH, CHUNK), jnp.int32),
        pltpu.VMEM((NCH, CHUNK), jnp.int32),
        pltpu.VMEM((CHUNK, DH), jnp.float32),
        pltpu.VMEM((CHUNK, DH), jnp.float32),
        pltpu.VMEM_SHARED((NPAD, DH), jnp.float32),
        pltpu.SemaphoreType.DMA,
        pltpu.SemaphoreType.DMA,
    ],
    compiler_params=pltpu.CompilerParams(use_tc_tiling_on_sc=False),
)
def _agg_kernel(ya_hbm, yb_hbm, src_hbm, dst_hbm,
                z0a_hbm, z0b_hbm, z1a_hbm, z1b_hbm,
                idxs_v, idxd_v, rows_a, rows_b, z_sp, sem_a, sem_b):
    cid = lax.axis_index("c")
    sid = lax.axis_index("s")
    w = cid * NS + sid
    pltpu.sync_copy(src_hbm.at[w], idxs_v)
    pltpu.sync_copy(dst_hbm.at[w], idxd_v)
    zv = jnp.zeros((16,), jnp.float32)

    for h, y_hbm in ((0, ya_hbm), (1, yb_hbm)):
        @pl.loop(0, CHUNK)
        def _(i):
            for jj in range(DH // 16):
                rows_a[i, pl.ds(jj * 16, 16)] = zv

        @pl.loop(0, RPT // CHUNK)
        def _(r):
            pltpu.sync_copy(
                rows_a, z_sp.at[pl.ds(sid * RPT + r * CHUNK, CHUNK)])

        plsc.subcore_barrier()

        pltpu.async_copy(y_hbm.at[idxs_v.at[0]], rows_a, sem_a)

        @pl.loop(0, NCH // 2)
        def _(j):
            c0 = 2 * j
            gb = pltpu.async_copy(y_hbm.at[idxs_v.at[c0 + 1]], rows_b, sem_b)
            pltpu.make_async_copy(
                y_hbm.at[idxs_v.at[c0]], rows_a, sem_a).wait()
            pltpu.sync_copy(rows_a, z_sp.at[idxd_v.at[c0]], add=True)

            @pl.when(c0 + 2 < NCH)
            def _():
                pltpu.async_copy(y_hbm.at[idxs_v.at[c0 + 2]], rows_a, sem_a)

            gb.wait()
            pltpu.sync_copy(rows_b, z_sp.at[idxd_v.at[c0 + 1]], add=True)

        plsc.subcore_barrier()

        zc0 = z0a_hbm if h == 0 else z0b_hbm
        zc1 = z1a_hbm if h == 0 else z1b_hbm

        @pl.when(cid == 0)
        def _():
            pltpu.sync_copy(z_sp.at[pl.ds(sid * RPT, RPT)],
                            zc0.at[pl.ds(sid * RPT, RPT)])

        @pl.when(cid == 1)
        def _():
            pltpu.sync_copy(z_sp.at[pl.ds(sid * RPT, RPT)],
                            zc1.at[pl.ds(sid * RPT, RPT)])

        plsc.subcore_barrier()


def _prep_body(degs_ref, x_ref, w_ref, ya_ref, yb_ref, dinv_ref):
    d = degs_ref[:, 0] + degs_ref[:, 1] + 1.0
    dinv = lax.rsqrt(d)[:, None]
    dinv_ref[...] = jnp.broadcast_to(dinv, x_ref.shape)
    y = dinv * jnp.dot(x_ref[...], w_ref[...],
                       preferred_element_type=jnp.float32)
    ya_ref[...] = y[:, :DH]
    yb_ref[...] = y[:, DH:]


def _post_body(layer, z0a_ref, z0b_ref, z1a_ref, z1b_ref, ya_ref, yb_ref,
               xin_ref, dinv_ref, b_ref, g_ref, bt_ref, wn_ref,
               h_ref, yan_ref, ybn_ref):
    dinv = dinv_ref[...]
    y = jnp.concatenate([ya_ref[...], yb_ref[...]], axis=1)
    z = jnp.concatenate([z0a_ref[...] + z1a_ref[...],
                         z0b_ref[...] + z1b_ref[...]], axis=1)
    h = dinv * (z + y) + b_ref[...]
    h = jnp.where(jnp.isnan(h), jnp.float32(0.0), h)
    h = jnp.where(jnp.isinf(h) & (h > 0), jnp.float32(EPS), h)
    h = jnp.where(jnp.isinf(h) & (h < 0), jnp.float32(-EPS), h)
    mu = jnp.mean(h, axis=-1, keepdims=True)
    var = jnp.mean((h - mu) ** 2, axis=-1, keepdims=True)
    h = (h - mu) / jnp.sqrt(var + LN_EPS) * g_ref[...] + bt_ref[...]
    if layer > 0:
        h = h + xin_ref[...]
    if layer < LAYERS - 1:
        h = jax.nn.relu(h)
    h_ref[...] = h
    if layer < LAYERS - 1:
        yn = dinv * jnp.dot(h, wn_ref[...],
                            preferred_element_type=jnp.float32)
        yan_ref[...] = yn[:, :DH]
        ybn_ref[...] = yn[:, DH:]


_BN = 1000


def _tc_prep(degs, x, w0):
    return pl.pallas_call(
        _prep_body,
        grid=(N // _BN,),
        in_specs=[
            pl.BlockSpec((_BN, 2), lambda i: (i, 0)),
            pl.BlockSpec((_BN, D), lambda i: (i, 0)),
            pl.BlockSpec((D, D), lambda i: (0, 0)),
        ],
        out_specs=[
            pl.BlockSpec((_BN, DH), lambda i: (i, 0)),
            pl.BlockSpec((_BN, DH), lambda i: (i, 0)),
            pl.BlockSpec((_BN, D), lambda i: (i, 0)),
        ],
        out_shape=[
            jax.ShapeDtypeStruct((N, DH), jnp.float32),
            jax.ShapeDtypeStruct((N, DH), jnp.float32),
            jax.ShapeDtypeStruct((N, D), jnp.float32),
        ],
    )(degs, x, w0)


def _tc_post(layer, z0a, z0b, z1a, z1b, ya, yb, xin, dinv2d, bl, gl, btl, wn):
    last = layer == LAYERS - 1
    if last:
        def body(z0ar, z0br, z1ar, z1br, yar, ybr, xr, dr, br, gr, btr, wr,
                 hr):
            _post_body(layer, z0ar, z0br, z1ar, z1br, yar, ybr, xr, dr, br,
                       gr, btr, wr, hr, None, None)
        out_specs = [pl.BlockSpec((_BN, D), lambda i: (i, 0))]
        out_shape = [jax.ShapeDtypeStruct((N, D), jnp.float32)]
    else:
        body = functools.partial(_post_body, layer)
        out_specs = [
            pl.BlockSpec((_BN, D), lambda i: (i, 0)),
            pl.BlockSpec((_BN, DH), lambda i: (i, 0)),
            pl.BlockSpec((_BN, DH), lambda i: (i, 0)),
        ]
        out_shape = [
            jax.ShapeDtypeStruct((N, D), jnp.float32),
            jax.ShapeDtypeStruct((N, DH), jnp.float32),
            jax.ShapeDtypeStruct((N, DH), jnp.float32),
        ]
    res = pl.pallas_call(
        body,
        grid=(N // _BN,),
        in_specs=[
            pl.BlockSpec((_BN, DH), lambda i: (i, 0)),
            pl.BlockSpec((_BN, DH), lambda i: (i, 0)),
            pl.BlockSpec((_BN, DH), lambda i: (i, 0)),
            pl.BlockSpec((_BN, DH), lambda i: (i, 0)),
            pl.BlockSpec((_BN, DH), lambda i: (i, 0)),
            pl.BlockSpec((_BN, DH), lambda i: (i, 0)),
            pl.BlockSpec((_BN, D), lambda i: (i, 0)),
            pl.BlockSpec((_BN, D), lambda i: (i, 0)),
            pl.BlockSpec((1, D), lambda i: (0, 0)),
            pl.BlockSpec((1, D), lambda i: (0, 0)),
            pl.BlockSpec((1, D), lambda i: (0, 0)),
            pl.BlockSpec((D, D), lambda i: (0, 0)),
        ],
        out_specs=out_specs,
        out_shape=out_shape,
    )(z0a, z0b, z1a, z1b, ya, yb, xin, dinv2d, bl, gl, btl, wn)
    return res if not last else (res[0], None, None)


@jax.jit
def kernel(x, edge_index, W, b, gamma, beta):
    npad_e = EP - E
    src_r = jnp.concatenate(
        [edge_index[0], jnp.zeros((npad_e,), jnp.int32)]
    ).reshape(NW, NCH, CHUNK)
    dst_r = jnp.concatenate(
        [edge_index[1], jnp.full((npad_e,), NPAD - 1, jnp.int32)]
    ).reshape(NW, NCH, CHUNK)
    zeros1d = jnp.zeros((RPT,), jnp.float32)
    ones_c = jnp.ones((CHUNK,), jnp.float32)

    d0, d1 = _deg_kernel(dst_r, zeros1d, ones_c)
    degs = jnp.stack([d0[:N], d1[:N]], axis=1)
    ya, yb, dinv2d = _tc_prep(degs, x, W[0])

    h = x
    for l in range(LAYERS):
        z0a, z0b, z1a, z1b = _agg_kernel(ya, yb, src_r, dst_r)
        wn = W[l + 1] if l < LAYERS - 1 else W[0]
        h, ya, yb = _tc_post(l, z0a, z0b, z1a, z1b, ya, yb, h, dinv2d,
                             b[l].reshape(1, D), gamma[l].reshape(1, D),
                             beta[l].reshape(1, D), wn)
    return h
